# baseline (speedup 1.0000x reference)
import functools

import jax
import jax.numpy as jnp
from jax import lax
from jax.experimental import pallas as pl
from jax.experimental.pallas import tpu as pltpu
from jax.experimental.pallas import tpu_sc as plsc

NC, NS = 2, 16
NW = NC * NS
L = 16
C = 2048
ROWS = 4 * 4096
ROWS_PER_W = ROWS // NW
CHUNK = 8
NCHUNK = ROWS_PER_W // CHUNK
GROUPS = C // L


def _shuffle_body(x_hbm, idx_hbm, out_hbm, idx_v, in_v, out_v):
    wid = lax.axis_index("s") * NC + lax.axis_index("c")
    pltpu.sync_copy(idx_hbm, idx_v)

    def chunk_body(ci, carry):
        base = (wid * NCHUNK + ci) * (CHUNK * C)
        pltpu.sync_copy(x_hbm.at[pl.ds(base, CHUNK * C)], in_v)

        def g_body(g, carry2):
            col = idx_v[pl.ds(g * L, L)]
            for r in range(CHUNK):
                v = plsc.load_gather(in_v, [col + r * C])
                out_v[pl.ds(r * C + g * L, L)] = v
            return carry2

        lax.fori_loop(0, GROUPS, g_body, 0)
        pltpu.sync_copy(out_v, out_hbm.at[pl.ds(base, CHUNK * C)])
        return carry

    lax.fori_loop(0, NCHUNK, chunk_body, 0)


_shuffle = functools.partial(
    pl.kernel,
    out_type=jax.ShapeDtypeStruct((ROWS * C,), jnp.float32),
    mesh=plsc.VectorSubcoreMesh(
        core_axis_name="c", subcore_axis_name="s",
        num_cores=NC, num_subcores=NS,
    ),
    scratch_types=[
        pltpu.VMEM((C,), jnp.int32),
        pltpu.VMEM((CHUNK * C,), jnp.float32),
        pltpu.VMEM((CHUNK * C,), jnp.float32),
    ],
    compiler_params=pltpu.CompilerParams(needs_layout_passes=False),
)(_shuffle_body)


def kernel(x, forward_shuffle_idx):
    out_flat = _shuffle(x.reshape(-1), forward_shuffle_idx)
    return out_flat.reshape(x.shape)


# speedup vs baseline: 2.0418x; 2.0418x over previous
import functools

import jax
import jax.numpy as jnp
from jax import lax
from jax.experimental import pallas as pl
from jax.experimental.pallas import tpu as pltpu
from jax.experimental.pallas import tpu_sc as plsc

NC, NS = 2, 16
NW = NC * NS
L = 16
C = 2048
ROWS = 4 * 4096
ROWS_PER_W = ROWS // NW
CHUNK = 8
CW = CHUNK * C
NCHUNK = ROWS_PER_W // CHUNK
GROUPS = C // L


def _permute_chunk(in_ref, out_ref, idx_v):
    @plsc.parallel_loop(0, GROUPS, unroll=2)
    def _(g):
        col = idx_v[pl.ds(g * L, L)]
        for r in range(CHUNK):
            v = plsc.load_gather(in_ref, [col + r * C])
            out_ref[pl.ds(r * C + g * L, L)] = v


def _shuffle_body(x_hbm, idx_hbm, out_hbm,
                  idx_v, in0, in1, out0, out1, si0, si1, so0, so1):
    wid = lax.axis_index("s") * NC + lax.axis_index("c")
    pltpu.sync_copy(idx_hbm, idx_v)
    base = wid * ROWS_PER_W * C
    ins, outs, sis, sos = (in0, in1), (out0, out1), (si0, si1), (so0, so1)

    def hbm_in(ci):
        return x_hbm.at[pl.ds(base + ci * CW, CW)]

    def hbm_out(ci):
        return out_hbm.at[pl.ds(base + ci * CW, CW)]

    pltpu.async_copy(hbm_in(0), in0, si0)
    pltpu.async_copy(hbm_in(1), in1, si1)

    nhalf = NCHUNK // 2

    def body(i, carry):
        for b in range(2):
            ci = 2 * i + b
            pltpu.make_async_copy(hbm_in(ci), ins[b], sis[b]).wait()

            @pl.when(i > 0)
            def _():
                pltpu.make_async_copy(outs[b], hbm_out(ci), sos[b]).wait()

            _permute_chunk(ins[b], outs[b], idx_v)
            pltpu.async_copy(outs[b], hbm_out(ci), sos[b])

            @pl.when(i < nhalf - 1)
            def _():
                pltpu.async_copy(hbm_in(ci + 2), ins[b], sis[b])
        return carry

    lax.fori_loop(0, nhalf, body, 0)
    pltpu.make_async_copy(out0, hbm_out(0), so0).wait()
    pltpu.make_async_copy(out1, hbm_out(1), so1).wait()


_shuffle = functools.partial(
    pl.kernel,
    out_type=jax.ShapeDtypeStruct((ROWS * C,), jnp.float32),
    mesh=plsc.VectorSubcoreMesh(
        core_axis_name="c", subcore_axis_name="s",
        num_cores=NC, num_subcores=NS,
    ),
    scratch_types=[
        pltpu.VMEM((C,), jnp.int32),
        pltpu.VMEM((CW,), jnp.float32),
        pltpu.VMEM((CW,), jnp.float32),
        pltpu.VMEM((CW,), jnp.float32),
        pltpu.VMEM((CW,), jnp.float32),
        pltpu.SemaphoreType.DMA,
        pltpu.SemaphoreType.DMA,
        pltpu.SemaphoreType.DMA,
        pltpu.SemaphoreType.DMA,
    ],
    compiler_params=pltpu.CompilerParams(needs_layout_passes=False),
)(_shuffle_body)


def kernel(x, forward_shuffle_idx):
    out_flat = _shuffle(x.reshape(-1), forward_shuffle_idx)
    return out_flat.reshape(x.shape)


# speedup vs baseline: 5.9592x; 2.9186x over previous
import functools

import jax
import jax.numpy as jnp
from jax import lax
from jax.experimental import pallas as pl
from jax.experimental.pallas import tpu as pltpu
from jax.experimental.pallas import tpu_sc as plsc

NC, NS = 2, 16
NW = NC * NS
L = 16
B, S, C = 4, 4096, 2048
ROWS = B * S
ROWS_PER_W = ROWS // NW
WPB = NW // B
CHUNK = 8
NCHUNK = ROWS_PER_W // CHUNK
GROUPS = C // L


def _permute_chunk(in_ref, out_ref, idx_v):
    @plsc.parallel_loop(0, GROUPS, unroll=2)
    def _(g):
        col = idx_v[pl.ds(g * L, L)]
        for r in range(CHUNK):
            row = jnp.full((L,), r, jnp.int32)
            v = plsc.load_gather(in_ref, [row, col])
            out_ref[r, pl.ds(g * L, L)] = v


def _shuffle_body(x_hbm, idx_hbm, out_hbm,
                  idx_v, in0, in1, out0, out1, si0, si1, so0, so1):
    wid = lax.axis_index("s") * NC + lax.axis_index("c")
    pltpu.sync_copy(idx_hbm, idx_v)
    b = wid // WPB
    row0 = (wid % WPB) * ROWS_PER_W
    ins, outs, sis, sos = (in0, in1), (out0, out1), (si0, si1), (so0, so1)

    def hbm_in(ci):
        return x_hbm.at[b, pl.ds(row0 + ci * CHUNK, CHUNK), :]

    def hbm_out(ci):
        return out_hbm.at[b, pl.ds(row0 + ci * CHUNK, CHUNK), :]

    pltpu.async_copy(hbm_in(0), in0, si0)
    pltpu.async_copy(hbm_in(1), in1, si1)

    nhalf = NCHUNK // 2

    def body(i, carry):
        for s in range(2):
            ci = 2 * i + s
            pltpu.make_async_copy(hbm_in(ci), ins[s], sis[s]).wait()

            @pl.when(i > 0)
            def _():
                pltpu.make_async_copy(outs[s], hbm_out(ci), sos[s]).wait()

            _permute_chunk(ins[s], outs[s], idx_v)
            pltpu.async_copy(outs[s], hbm_out(ci), sos[s])

            @pl.when(i < nhalf - 1)
            def _():
                pltpu.async_copy(hbm_in(ci + 2), ins[s], sis[s])
        return carry

    lax.fori_loop(0, nhalf, body, 0)
    pltpu.make_async_copy(out0, hbm_out(0), so0).wait()
    pltpu.make_async_copy(out1, hbm_out(1), so1).wait()


_shuffle = functools.partial(
    pl.kernel,
    out_type=jax.ShapeDtypeStruct((B, S, C), jnp.float32),
    mesh=plsc.VectorSubcoreMesh(
        core_axis_name="c", subcore_axis_name="s",
        num_cores=NC, num_subcores=NS,
    ),
    scratch_types=[
        pltpu.VMEM((C,), jnp.int32),
        pltpu.VMEM((CHUNK, C), jnp.float32),
        pltpu.VMEM((CHUNK, C), jnp.float32),
        pltpu.VMEM((CHUNK, C), jnp.float32),
        pltpu.VMEM((CHUNK, C), jnp.float32),
        pltpu.SemaphoreType.DMA,
        pltpu.SemaphoreType.DMA,
        pltpu.SemaphoreType.DMA,
        pltpu.SemaphoreType.DMA,
    ],
    compiler_params=pltpu.CompilerParams(needs_layout_passes=False),
)(_shuffle_body)


def kernel(x, forward_shuffle_idx):
    return _shuffle(x, forward_shuffle_idx)


# speedup vs baseline: 6.1854x; 1.0380x over previous
import functools

import jax
import jax.numpy as jnp
from jax import lax
from jax.experimental import pallas as pl
from jax.experimental.pallas import tpu as pltpu
from jax.experimental.pallas import tpu_sc as plsc

NC, NS = 2, 16
NW = NC * NS
L = 16
B, S, C = 4, 4096, 2048
ROWS = B * S
ROWS_PER_W = ROWS // NW
WPB = NW // B
CHUNK = 4
NBUF = 4
NCHUNK = ROWS_PER_W // CHUNK
GROUPS = C // L


def _permute_chunk(in_ref, out_ref, idx_v):
    @plsc.parallel_loop(0, GROUPS, unroll=4)
    def _(g):
        col = idx_v[pl.ds(g * L, L)]
        for r in range(CHUNK):
            row = jnp.full((L,), r, jnp.int32)
            v = plsc.load_gather(in_ref, [row, col])
            out_ref[r, pl.ds(g * L, L)] = v


def _shuffle_body(x_hbm, idx_hbm, out_hbm, idx_v, ins, outs, sis, sos):
    wid = lax.axis_index("s") * NC + lax.axis_index("c")
    pltpu.sync_copy(idx_hbm, idx_v)
    b = wid // WPB
    row0 = (wid % WPB) * ROWS_PER_W

    def hbm_in(ci):
        return x_hbm.at[b, pl.ds(row0 + ci * CHUNK, CHUNK), :]

    def hbm_out(ci):
        return out_hbm.at[b, pl.ds(row0 + ci * CHUNK, CHUNK), :]

    for s in range(NBUF):
        pltpu.async_copy(hbm_in(s), ins[s], sis[s])

    nstep = NCHUNK // NBUF

    def body(i, carry):
        for s in range(NBUF):
            ci = NBUF * i + s
            pltpu.make_async_copy(hbm_in(ci), ins[s], sis[s]).wait()

            @pl.when(i > 0)
            def _():
                pltpu.make_async_copy(outs[s], hbm_out(ci), sos[s]).wait()

            _permute_chunk(ins[s], outs[s], idx_v)
            pltpu.async_copy(outs[s], hbm_out(ci), sos[s])

            @pl.when(i < nstep - 1)
            def _():
                pltpu.async_copy(hbm_in(ci + NBUF), ins[s], sis[s])
        return carry

    lax.fori_loop(0, nstep, body, 0)
    for s in range(NBUF):
        pltpu.make_async_copy(outs[s], hbm_out(s), sos[s]).wait()


_shuffle = functools.partial(
    pl.kernel,
    out_type=jax.ShapeDtypeStruct((B, S, C), jnp.float32),
    mesh=plsc.VectorSubcoreMesh(
        core_axis_name="c", subcore_axis_name="s",
        num_cores=NC, num_subcores=NS,
    ),
    scratch_types=[
        pltpu.VMEM((C,), jnp.int32),
        tuple(pltpu.VMEM((CHUNK, C), jnp.float32) for _ in range(NBUF)),
        tuple(pltpu.VMEM((CHUNK, C), jnp.float32) for _ in range(NBUF)),
        tuple(pltpu.SemaphoreType.DMA for _ in range(NBUF)),
        tuple(pltpu.SemaphoreType.DMA for _ in range(NBUF)),
    ],
    compiler_params=pltpu.CompilerParams(needs_layout_passes=False),
)(_shuffle_body)


def kernel(x, forward_shuffle_idx):
    return _shuffle(x, forward_shuffle_idx)
